# baseline (speedup 1.0000x reference)
import jax
import jax.numpy as jnp
from jax import lax
from jax.experimental import pallas as pl
from jax.experimental.pallas import tpu as pltpu
from jax.experimental.pallas import tpu_sc as plsc

D = 32
H = D // 2
B = 16384
LAM = 1e-5

NC = 2
NS = 16
NW = NC * NS
PER_W = B // NW
SUB = 64
NSUB = PER_W // SUB
IG = PER_W // 16

QB = 4096
NBLK = 62
QCAP = NBLK * QB
QSTART = (0, 62, 124, 183)
B1 = QSTART[1] * QB
B2 = QSTART[2] * QB
B3 = QSTART[3] * QB


def _tp_body(x0, x1, x2, x3, out_ref):
    for q, xq in enumerate((x0, x1, x2, x3)):
        t = xq[...].T
        lo = lax.bitcast_convert_type(
            t[:, 0:H].astype(jnp.bfloat16), jnp.int16).astype(jnp.int32)
        hi = lax.bitcast_convert_type(
            t[:, H:D].astype(jnp.bfloat16), jnp.int16).astype(jnp.int32)
        out_ref[:, q * H:(q + 1) * H] = (lo & 0xFFFF) | (hi << 16)


def _pack(tt):
    return pl.pallas_call(
        _tp_body,
        grid=(NBLK,),
        in_specs=[
            pl.BlockSpec((D, QB), lambda b, q=q: (0, QSTART[q] + b))
            for q in range(4)
        ],
        out_specs=pl.BlockSpec((QB, 4 * H), lambda b: (b, 0)),
        out_shape=jax.ShapeDtypeStruct((QCAP, 4 * H), jnp.int32),
    )(tt, tt, tt, tt)


def _rowoff(e):
    q1 = jnp.where(e >= B1, 1, 0)
    q2 = jnp.where(e >= B2, 1, 0)
    q3 = jnp.where(e >= B3, 1, 0)
    row = e - q1 * B1 - q2 * (B2 - B1) - q3 * (B3 - B2)
    off = (q1 + q2 + q3) * H
    return row, off


def _sc_body(h_hbm, r_hbm, pos_hbm, neg_hbm, tb_re, tb_im, rel_re, rel_im,
             hre_out, him_out, pre_out, pim_out, nre_out, nim_out,
             rr0_out, rr1_out, ri0_out, ri1_out,
             h_v, r_v, p_v, n_v,
             hrow, prow, nrow, hoff, poff, noff,
             hrb, hib, prb, pib, nrb, nib, rrb, rib,
             hre_v, him_v, pre_v, pim_v, nre_v, nim_v,
             rr0_v, rr1_v, ri0_v, ri1_v, sem):
    wid = lax.axis_index("s") * NC + lax.axis_index("c")
    base = wid * PER_W

    icps = [
        pltpu.async_copy(h_hbm.at[pl.ds(base, PER_W)], h_v, sem),
        pltpu.async_copy(r_hbm.at[pl.ds(base, PER_W)], r_v, sem),
        pltpu.async_copy(pos_hbm.at[pl.ds(base, PER_W)], p_v, sem),
        pltpu.async_copy(neg_hbm.at[pl.ds(base, PER_W)], n_v, sem),
    ]
    for cp in icps:
        cp.wait()

    def prep_body(g, carry):
        sl = pl.ds(g * 16, 16)
        row, off = _rowoff(h_v[sl])
        hrow[sl] = row
        hoff[sl] = off
        row, off = _rowoff(p_v[sl])
        prow[sl] = row
        poff[sl] = off
        row, off = _rowoff(n_v[sl])
        nrow[sl] = row
        noff[sl] = off
        return carry

    lax.fori_loop(0, IG, prep_body, 0)

    def subchunk_body(s, carry):
        sl = pl.ds(s * SUB, SUB)
        cps = [
            pltpu.async_copy(tb_re.at[hrow.at[sl]], hrb, sem),
            pltpu.async_copy(tb_im.at[hrow.at[sl]], hib, sem),
            pltpu.async_copy(rel_re.at[r_v.at[sl]], rrb, sem),
            pltpu.async_copy(rel_im.at[r_v.at[sl]], rib, sem),
            pltpu.async_copy(tb_re.at[prow.at[sl]], prb, sem),
            pltpu.async_copy(tb_im.at[prow.at[sl]], pib, sem),
            pltpu.async_copy(tb_re.at[nrow.at[sl]], nrb, sem),
            pltpu.async_copy(tb_im.at[nrow.at[sl]], nib, sem),
        ]
        for cp in cps:
            cp.wait()

        def group_body(g, carry2):
            i0 = s * SUB + g * 16
            hof = hoff[pl.ds(i0, 16)]
            pof = poff[pl.ds(i0, 16)]
            nof = noff[pl.ds(i0, 16)]
            for k in range(16):
                j = g * 16 + k
                i = i0 + k
                ho = hof[k]
                po = pof[k]
                no = nof[k]
                hre_v[i, :] = hrb[j, pl.ds(ho, 16)]
                him_v[i, :] = hib[j, pl.ds(ho, 16)]
                pre_v[i, :] = prb[j, pl.ds(po, 16)]
                pim_v[i, :] = pib[j, pl.ds(po, 16)]
                nre_v[i, :] = nrb[j, pl.ds(no, 16)]
                nim_v[i, :] = nib[j, pl.ds(no, 16)]
                rr0_v[i, :] = rrb[j, pl.ds(0, 16)]
                rr1_v[i, :] = rrb[j, pl.ds(16, 16)]
                ri0_v[i, :] = rib[j, pl.ds(0, 16)]
                ri1_v[i, :] = rib[j, pl.ds(16, 16)]
            return carry2

        return lax.fori_loop(0, SUB // 16, group_body, carry)

    lax.fori_loop(0, NSUB, subchunk_body, 0)
    pltpu.sync_copy(hre_v, hre_out.at[pl.ds(base, PER_W)])
    pltpu.sync_copy(him_v, him_out.at[pl.ds(base, PER_W)])
    pltpu.sync_copy(pre_v, pre_out.at[pl.ds(base, PER_W)])
    pltpu.sync_copy(pim_v, pim_out.at[pl.ds(base, PER_W)])
    pltpu.sync_copy(nre_v, nre_out.at[pl.ds(base, PER_W)])
    pltpu.sync_copy(nim_v, nim_out.at[pl.ds(base, PER_W)])
    pltpu.sync_copy(rr0_v, rr0_out.at[pl.ds(base, PER_W)])
    pltpu.sync_copy(rr1_v, rr1_out.at[pl.ds(base, PER_W)])
    pltpu.sync_copy(ri0_v, ri0_out.at[pl.ds(base, PER_W)])
    pltpu.sync_copy(ri1_v, ri1_out.at[pl.ds(base, PER_W)])


_sc_call = pl.kernel(
    _sc_body,
    mesh=plsc.VectorSubcoreMesh(core_axis_name="c", subcore_axis_name="s"),
    compiler_params=pltpu.CompilerParams(use_tc_tiling_on_sc=False),
    out_type=(
        [jax.ShapeDtypeStruct((B, H), jnp.int32)] * 6
        + [jax.ShapeDtypeStruct((B, H), jnp.float32)] * 4
    ),
    scratch_types=(
        [pltpu.VMEM((PER_W,), jnp.int32)] * 10
        + [pltpu.VMEM((SUB, 4 * H), jnp.int32)] * 6
        + [pltpu.VMEM((SUB, D), jnp.float32)] * 2
        + [pltpu.VMEM((PER_W, H), jnp.int32)] * 6
        + [pltpu.VMEM((PER_W, H), jnp.float32)] * 4
        + [pltpu.SemaphoreType.DMA]
    ),
)


def _dec(v):
    lo = lax.bitcast_convert_type(v << 16, jnp.float32)
    hi = lax.bitcast_convert_type(v & jnp.int32(-65536), jnp.float32)
    return lo, hi


def _tc_body(hre_ref, him_ref, pre_ref, pim_ref, nre_ref, nim_ref,
             rr0_ref, rr1_ref, ri0_ref, ri1_ref, out_ref):
    h0, h1 = _dec(hre_ref[...])
    hi0, hi1 = _dec(him_ref[...])
    p0, p1 = _dec(pre_ref[...])
    pi0, pi1 = _dec(pim_ref[...])
    n0, n1 = _dec(nre_ref[...])
    ni0, ni1 = _dec(nim_ref[...])
    r0 = rr0_ref[...]
    r1 = rr1_ref[...]
    ri0 = ri0_ref[...]
    ri1 = ri1_ref[...]
    a0 = h0 * r0 - hi0 * ri0
    b0 = hi0 * r0 + h0 * ri0
    a1 = h1 * r1 - hi1 * ri1
    b1 = hi1 * r1 + h1 * ri1
    part = (a0 * (n0 - p0) + b0 * (ni0 - pi0)
            + a1 * (n1 - p1) + b1 * (ni1 - pi1))
    lane = lax.broadcasted_iota(jnp.int32, (128, 8), 0)
    col = lax.broadcasted_iota(jnp.int32, (128, 8), 1)
    m = jnp.where(lane // 16 == col, 1.0, 0.0)
    d = jax.lax.dot_general(part, m, (((1,), (0,)), ((), ())),
                            preferred_element_type=jnp.float32)
    nls = jnp.log1p(jnp.exp(-jnp.abs(d))) - jnp.minimum(d, 0.0)
    l2 = (jnp.sum(h0 * h0) + jnp.sum(h1 * h1) + jnp.sum(hi0 * hi0)
          + jnp.sum(hi1 * hi1) + jnp.sum(p0 * p0) + jnp.sum(p1 * p1)
          + jnp.sum(pi0 * pi0) + jnp.sum(pi1 * pi1) + jnp.sum(n0 * n0)
          + jnp.sum(n1 * n1) + jnp.sum(ni0 * ni0) + jnp.sum(ni1 * ni1)
          + jnp.sum(r0 * r0) + jnp.sum(r1 * r1) + jnp.sum(ri0 * ri0)
          + jnp.sum(ri1 * ri1))
    out_ref[0, 0] = jnp.sum(nls) / B + (LAM * 0.5 / B) * l2


def kernel(h, r, pos_t, neg_t, ent_re, ent_im, rel_re, rel_im):
    tb_re = _pack(ent_re.T)
    tb_im = _pack(ent_im.T)
    outs = _sc_call(h, r, pos_t, neg_t, tb_re, tb_im, rel_re, rel_im)
    outs = [o.reshape(B // 8, 128) for o in outs]
    loss = pl.pallas_call(
        _tc_body,
        out_shape=jax.ShapeDtypeStruct((1, 1), jnp.float32),
        out_specs=pl.BlockSpec(memory_space=pltpu.SMEM),
    )(*outs)
    return loss[0, 0]


# speedup vs baseline: 1.4428x; 1.4428x over previous
import jax
import jax.numpy as jnp
from jax import lax
from jax.experimental import pallas as pl
from jax.experimental.pallas import tpu as pltpu
from jax.experimental.pallas import tpu_sc as plsc

D = 32
N = 1000000
B = 16384
LAM = 1e-5

NC = 2
NS = 16
NW = NC * NS
PER_W = B // NW
SUB = 64
NSUB = PER_W // SUB
IG = PER_W // 16

QB = 4096
NBLK = 62
QCAP = NBLK * QB
QSTART = (0, 62, 124, 183)
B1 = QSTART[1] * QB
B2 = QSTART[2] * QB
B3 = QSTART[3] * QB


def _tp_body(a0, a1, a2, a3, b0, b1, b2, b3, oa_ref, ob_ref):
    for q, xq in enumerate((a0, a1, a2, a3)):
        oa_ref[:, q * D:(q + 1) * D] = xq[...].T
    for q, xq in enumerate((b0, b1, b2, b3)):
        ob_ref[:, q * D:(q + 1) * D] = xq[...].T


def _tp_call(ta, tb):
    spec = [
        pl.BlockSpec((D, QB), lambda b, q=q: (0, QSTART[q] + b))
        for q in range(4)
    ]
    ospec = pl.BlockSpec((QB, 4 * D), lambda b: (b, 0))
    oshape = jax.ShapeDtypeStruct((QCAP, 4 * D), jnp.float32)
    return pl.pallas_call(
        _tp_body,
        grid=(NBLK,),
        in_specs=spec + spec,
        out_specs=[ospec, ospec],
        out_shape=[oshape, oshape],
    )(ta, ta, ta, ta, tb, tb, tb, tb)


def _rowoff(e):
    q1 = jnp.where(e >= B1, 1, 0)
    q2 = jnp.where(e >= B2, 1, 0)
    q3 = jnp.where(e >= B3, 1, 0)
    row = e - q1 * B1 - q2 * (B2 - B1) - q3 * (B3 - B2)
    off = (q1 + q2 + q3) * D
    return row, off


def _sc_body(h_hbm, r_hbm, pos_hbm, neg_hbm, tb_re, tb_im, rel_re, rel_im,
             part_out, l2_out,
             h_v, r_v, p_v, n_v,
             hrow, prow, nrow, hoff, poff, noff,
             hrb, hib, prb, pib, nrb, nib, rrb, rib,
             part_v, l2_v, sem):
    wid = lax.axis_index("s") * NC + lax.axis_index("c")
    base = wid * PER_W

    icps = [
        pltpu.async_copy(h_hbm.at[pl.ds(base, PER_W)], h_v, sem),
        pltpu.async_copy(r_hbm.at[pl.ds(base, PER_W)], r_v, sem),
        pltpu.async_copy(pos_hbm.at[pl.ds(base, PER_W)], p_v, sem),
        pltpu.async_copy(neg_hbm.at[pl.ds(base, PER_W)], n_v, sem),
    ]
    for cp in icps:
        cp.wait()

    def prep_body(g, carry):
        sl = pl.ds(g * 16, 16)
        row, off = _rowoff(h_v[sl])
        hrow[sl] = row
        hoff[sl] = off
        row, off = _rowoff(p_v[sl])
        prow[sl] = row
        poff[sl] = off
        row, off = _rowoff(n_v[sl])
        nrow[sl] = row
        noff[sl] = off
        return carry

    lax.fori_loop(0, IG, prep_body, 0)

    def subchunk_body(s, l2acc):
        sl = pl.ds(s * SUB, SUB)
        cps = [
            pltpu.async_copy(tb_re.at[hrow.at[sl]], hrb, sem),
            pltpu.async_copy(tb_im.at[hrow.at[sl]], hib, sem),
            pltpu.async_copy(rel_re.at[r_v.at[sl]], rrb, sem),
            pltpu.async_copy(rel_im.at[r_v.at[sl]], rib, sem),
            pltpu.async_copy(tb_re.at[prow.at[sl]], prb, sem),
            pltpu.async_copy(tb_im.at[prow.at[sl]], pib, sem),
            pltpu.async_copy(tb_re.at[nrow.at[sl]], nrb, sem),
            pltpu.async_copy(tb_im.at[nrow.at[sl]], nib, sem),
        ]
        for cp in cps:
            cp.wait()

        def group_body(g, l2a):
            i0 = s * SUB + g * 16
            hof = hoff[pl.ds(i0, 16)]
            pof = poff[pl.ds(i0, 16)]
            nof = noff[pl.ds(i0, 16)]
            for k in range(16):
                j = g * 16 + k
                ho = hof[k]
                po = pof[k]
                no = nof[k]
                h0 = hrb[j, pl.ds(ho, 16)]
                h1 = hrb[j, pl.ds(ho + 16, 16)]
                hi0 = hib[j, pl.ds(ho, 16)]
                hi1 = hib[j, pl.ds(ho + 16, 16)]
                r0 = rrb[j, pl.ds(0, 16)]
                r1 = rrb[j, pl.ds(16, 16)]
                ri0 = rib[j, pl.ds(0, 16)]
                ri1 = rib[j, pl.ds(16, 16)]
                p0 = prb[j, pl.ds(po, 16)]
                p1 = prb[j, pl.ds(po + 16, 16)]
                pi0 = pib[j, pl.ds(po, 16)]
                pi1 = pib[j, pl.ds(po + 16, 16)]
                n0 = nrb[j, pl.ds(no, 16)]
                n1 = nrb[j, pl.ds(no + 16, 16)]
                ni0 = nib[j, pl.ds(no, 16)]
                ni1 = nib[j, pl.ds(no + 16, 16)]
                a0 = h0 * r0 - hi0 * ri0
                b0 = hi0 * r0 + h0 * ri0
                a1 = h1 * r1 - hi1 * ri1
                b1 = hi1 * r1 + h1 * ri1
                part = (a0 * (n0 - p0) + b0 * (ni0 - pi0)
                        + a1 * (n1 - p1) + b1 * (ni1 - pi1))
                part_v[s * SUB + j, :] = part
                l2a = (l2a + h0 * h0 + h1 * h1 + hi0 * hi0 + hi1 * hi1
                       + r0 * r0 + r1 * r1 + ri0 * ri0 + ri1 * ri1
                       + p0 * p0 + p1 * p1 + pi0 * pi0 + pi1 * pi1
                       + n0 * n0 + n1 * n1 + ni0 * ni0 + ni1 * ni1)
            return l2a

        return lax.fori_loop(0, SUB // 16, group_body, l2acc)

    l2acc = lax.fori_loop(0, NSUB, subchunk_body, jnp.zeros((16,), jnp.float32))
    l2_v[...] = l2acc
    pltpu.sync_copy(part_v, part_out.at[pl.ds(base, PER_W)])
    pltpu.sync_copy(l2_v, l2_out.at[wid])


_sc_call = pl.kernel(
    _sc_body,
    mesh=plsc.VectorSubcoreMesh(core_axis_name="c", subcore_axis_name="s"),
    compiler_params=pltpu.CompilerParams(use_tc_tiling_on_sc=False),
    out_type=[
        jax.ShapeDtypeStruct((B, 16), jnp.float32),
        jax.ShapeDtypeStruct((NW, 16), jnp.float32),
    ],
    scratch_types=[
        pltpu.VMEM((PER_W,), jnp.int32),
        pltpu.VMEM((PER_W,), jnp.int32),
        pltpu.VMEM((PER_W,), jnp.int32),
        pltpu.VMEM((PER_W,), jnp.int32),
        pltpu.VMEM((PER_W,), jnp.int32),
        pltpu.VMEM((PER_W,), jnp.int32),
        pltpu.VMEM((PER_W,), jnp.int32),
        pltpu.VMEM((PER_W,), jnp.int32),
        pltpu.VMEM((PER_W,), jnp.int32),
        pltpu.VMEM((PER_W,), jnp.int32),
        pltpu.VMEM((SUB, 4 * D), jnp.float32),
        pltpu.VMEM((SUB, 4 * D), jnp.float32),
        pltpu.VMEM((SUB, 4 * D), jnp.float32),
        pltpu.VMEM((SUB, 4 * D), jnp.float32),
        pltpu.VMEM((SUB, 4 * D), jnp.float32),
        pltpu.VMEM((SUB, 4 * D), jnp.float32),
        pltpu.VMEM((SUB, D), jnp.float32),
        pltpu.VMEM((SUB, D), jnp.float32),
        pltpu.VMEM((PER_W, 16), jnp.float32),
        pltpu.VMEM((16,), jnp.float32),
        pltpu.SemaphoreType.DMA,
    ],
)


def _tc_body(part_ref, l2_ref, out_ref):
    x = part_ref[...]
    lane = lax.broadcasted_iota(jnp.int32, (128, 8), 0)
    col = lax.broadcasted_iota(jnp.int32, (128, 8), 1)
    m = jnp.where(lane // 16 == col, 1.0, 0.0)
    d = jax.lax.dot_general(x, m, (((1,), (0,)), ((), ())),
                            preferred_element_type=jnp.float32)
    nls = jnp.log1p(jnp.exp(-jnp.abs(d))) - jnp.minimum(d, 0.0)
    out_ref[0, 0] = jnp.sum(nls) / B + (LAM * 0.5 / B) * jnp.sum(l2_ref[...])


def kernel(h, r, pos_t, neg_t, ent_re, ent_im, rel_re, rel_im):
    tb_re, tb_im = _tp_call(ent_re.T, ent_im.T)
    part, l2p = _sc_call(h, r, pos_t, neg_t, tb_re, tb_im, rel_re, rel_im)
    loss = pl.pallas_call(
        _tc_body,
        out_shape=jax.ShapeDtypeStruct((1, 1), jnp.float32),
        out_specs=pl.BlockSpec(memory_space=pltpu.SMEM),
    )(part.reshape(B // 8, 128), l2p)
    return loss[0, 0]
